# Initial kernel scaffold; baseline (speedup 1.0000x reference)
import jax
import jax.numpy as jnp
from jax.experimental import pallas as pl


def kernel(predictions):
    raise NotImplementedError("write your pallas kernel here")


# speedup vs baseline: 1.3579x; 1.3579x over previous
import jax
import jax.numpy as jnp
from jax.experimental import pallas as pl
from jax.experimental.pallas import tpu as pltpu

_CONF = 0.5
_NMS = 50.0
_MAXL = 5
_NSTRIPS = 71.0
_NOFF = 72
_NCOLS = 76
_N = 20000
_C = 2000
_NCHUNK = _N // _C
_DEAD = -1e9


def _nms_kernel(pred_ref, kept_ref, keep_ref, num_ref, live_ref):
    kidx = jax.lax.broadcasted_iota(jnp.int32, (1, _NOFF), 1).astype(jnp.float32)
    lane = jax.lax.broadcasted_iota(jnp.int32, (1, _NCOLS), 1)

    def init_chunk(j, carry):
        b = j * _C
        p = pred_ref[pl.ds(b, _C), :]
        p0 = p[:, 0:1]
        p1 = p[:, 1:2]
        m = jnp.maximum(p0, p1)
        e0 = jnp.exp(p0 - m)
        e1 = jnp.exp(p1 - m)
        score = e1 / (e0 + e1)
        live_ref[pl.ds(b, _C), :] = jnp.where(score >= _CONF, score, _DEAD)
        return carry

    jax.lax.fori_loop(0, _NCHUNK, init_chunk, 0)
    num_ref[0] = 0

    def it(t, carry):
        def amax_chunk(j, c):
            mx, bi = c
            b = j * _C
            lv = live_ref[pl.ds(b, _C), :]
            ridx = jax.lax.broadcasted_iota(jnp.int32, (_C, 1), 0) + b
            cm = jnp.max(lv)
            ci = jnp.min(jnp.where(lv == cm, ridx, _N))
            take = cm > mx
            return (jnp.where(take, cm, mx), jnp.where(take, ci, bi))

        mx, bi = jax.lax.fori_loop(
            0, _NCHUNK, amax_chunk, (jnp.float32(-2e9), jnp.int32(0)))
        valid = mx > -1e8

        row = pred_ref[pl.ds(bi, 1), :]
        rb2 = row[:, 2:3]
        rb3 = row[:, 3:4]
        sb = jnp.clip(jnp.round(rb2 * _NSTRIPS), 0.0, _NSTRIPS)
        eb = jnp.clip(jnp.round(rb2 * _NSTRIPS + rb3 * _NSTRIPS - 1.0),
                      0.0, _NSTRIPS)
        xb = row[:, 4:] * 800.0

        def supp_chunk(j, c):
            b = j * _C
            p = pred_ref[pl.ds(b, _C), :]
            p2 = p[:, 2:3]
            p3 = p[:, 3:4]
            start = jnp.clip(jnp.round(p2 * _NSTRIPS), 0.0, _NSTRIPS)
            end = jnp.clip(jnp.round(p2 * _NSTRIPS + p3 * _NSTRIPS - 1.0),
                           0.0, _NSTRIPS)
            xs = p[:, 4:] * 800.0
            s = jnp.maximum(start, sb)
            e = jnp.minimum(end, eb)
            maskf = ((kidx >= s) & (kidx <= e)).astype(jnp.float32)
            diff = jnp.abs(xs - xb)
            cnt = jnp.sum(maskf, axis=1, keepdims=True)
            dist = (jnp.sum(diff * maskf, axis=1, keepdims=True)
                    / jnp.maximum(cnt, 1.0))
            ridx = jax.lax.broadcasted_iota(jnp.int32, (_C, 1), 0) + b
            supp = ((dist < _NMS) & (cnt > 0.0)) | (ridx == bi)
            lv = live_ref[pl.ds(b, _C), :]
            live_ref[pl.ds(b, _C), :] = jnp.where(supp & valid, _DEAD, lv)
            return c

        jax.lax.fori_loop(0, _NCHUNK, supp_chunk, 0)

        validf = valid.astype(jnp.float32)
        col3 = jnp.round(rb3 * _NSTRIPS) * validf
        out_row = jnp.where(lane == 3, col3, row * validf)
        kept_ref[pl.ds(t, 1), :] = out_row
        keep_ref[t] = jnp.where(valid, bi, jnp.int32(-1))
        num_ref[0] = num_ref[0] + valid.astype(jnp.int32)
        return carry

    jax.lax.fori_loop(0, _MAXL, it, 0)


def kernel(predictions):
    kept, keep, num = pl.pallas_call(
        _nms_kernel,
        out_shape=(
            jax.ShapeDtypeStruct((_MAXL, _NCOLS), jnp.float32),
            jax.ShapeDtypeStruct((_MAXL,), jnp.int32),
            jax.ShapeDtypeStruct((1,), jnp.int32),
        ),
        in_specs=[pl.BlockSpec(memory_space=pltpu.VMEM)],
        out_specs=(
            pl.BlockSpec(memory_space=pltpu.VMEM),
            pl.BlockSpec(memory_space=pltpu.SMEM),
            pl.BlockSpec(memory_space=pltpu.SMEM),
        ),
        scratch_shapes=[pltpu.VMEM((_N, 1), jnp.float32)],
    )(predictions)
    return kept, keep, num[0]


# speedup vs baseline: 5.0292x; 3.7037x over previous
import jax
import jax.numpy as jnp
from jax.experimental import pallas as pl
from jax.experimental.pallas import tpu as pltpu

_CONF = 0.5
_THR = 50.0 / 800.0
_MAXL = 5
_NSTRIPS = 71.0
_NROWBLK = 9
_NCOLS = 76
_N = 20000
_DEAD = -1e9


def _nms_kernel(pt_ref, pred_ref, kept_ref, keep_ref, num_ref, state_ref):
    p0 = pt_ref[0:1, :]
    p1 = pt_ref[1:2, :]
    m = jnp.maximum(p0, p1)
    e0 = jnp.exp(p0 - m)
    e1 = jnp.exp(p1 - m)
    score = e1 / (e0 + e1)
    state_ref[0:1, :] = jnp.where(score >= _CONF, score, _DEAD)
    p2 = pt_ref[2:3, :]
    p3 = pt_ref[3:4, :]
    state_ref[1:2, :] = jnp.clip(jnp.round(p2 * _NSTRIPS), 0.0, _NSTRIPS)
    state_ref[2:3, :] = jnp.clip(
        jnp.round(p2 * _NSTRIPS + p3 * _NSTRIPS - 1.0), 0.0, _NSTRIPS)
    num_ref[0] = 0

    lane76 = jax.lax.broadcasted_iota(jnp.int32, (1, _NCOLS), 1)

    def it(t, carry):
        live = state_ref[0:1, :]
        lidx = jax.lax.broadcasted_iota(jnp.int32, (1, _N), 1)
        mx = jnp.max(live)
        bi = jnp.min(jnp.where(live == mx, lidx, _N))
        valid = mx > -1e8
        sel = (lidx == bi).astype(jnp.float32)

        row = pred_ref[pl.ds(bi, 1), :]
        rb2 = row[:, 2:3]
        rb3 = row[:, 3:4]
        sb = jnp.clip(jnp.round(rb2 * _NSTRIPS), 0.0, _NSTRIPS)
        eb = jnp.clip(jnp.round(rb2 * _NSTRIPS + rb3 * _NSTRIPS - 1.0),
                      0.0, _NSTRIPS)

        start = state_ref[1:2, :]
        end = state_ref[2:3, :]
        s = jnp.maximum(start, sb)
        e = jnp.minimum(end, eb)
        cnt = jnp.maximum(e - s + 1.0, 0.0)

        acc = jnp.zeros((8, _N), jnp.float32)
        for r in range(_NROWBLK):
            xsr = pt_ref[pl.ds(8 + 8 * r, 8), :]
            xbr = jnp.sum(xsr * sel, axis=1, keepdims=True)
            kr = (jax.lax.broadcasted_iota(jnp.int32, (8, 1), 0)
                  + 8 * r).astype(jnp.float32)
            maskf = ((kr >= s) & (kr <= e)).astype(jnp.float32)
            acc = acc + jnp.abs(xsr - xbr) * maskf
        dist = jnp.sum(acc, axis=0, keepdims=True)

        supp = ((dist < cnt * _THR) & (cnt > 0.0)) | (lidx == bi)
        state_ref[0:1, :] = jnp.where(supp & valid, _DEAD, live)

        validf = valid.astype(jnp.float32)
        col3 = jnp.round(rb3 * _NSTRIPS) * validf
        out_row = jnp.where(lane76 == 3, col3, row * validf)
        kept_ref[pl.ds(t, 1), :] = out_row
        keep_ref[t] = jnp.where(valid, bi, jnp.int32(-1))
        num_ref[0] = num_ref[0] + valid.astype(jnp.int32)
        return carry

    jax.lax.fori_loop(0, _MAXL, it, 0)


def kernel(predictions):
    pt = predictions.T
    ptp = jnp.concatenate(
        [pt[:4], jnp.zeros((4, _N), jnp.float32), pt[4:]], axis=0)
    kept, keep, num = pl.pallas_call(
        _nms_kernel,
        out_shape=(
            jax.ShapeDtypeStruct((_MAXL, _NCOLS), jnp.float32),
            jax.ShapeDtypeStruct((_MAXL,), jnp.int32),
            jax.ShapeDtypeStruct((1,), jnp.int32),
        ),
        in_specs=[
            pl.BlockSpec(memory_space=pltpu.VMEM),
            pl.BlockSpec(memory_space=pltpu.VMEM),
        ],
        out_specs=(
            pl.BlockSpec(memory_space=pltpu.VMEM),
            pl.BlockSpec(memory_space=pltpu.SMEM),
            pl.BlockSpec(memory_space=pltpu.SMEM),
        ),
        scratch_shapes=[pltpu.VMEM((8, _N), jnp.float32)],
    )(ptp, predictions)
    return kept, keep, num[0]
